# baseline (speedup 1.0000x reference)
import jax
import jax.numpy as jnp
from jax.experimental import pallas as pl
from jax.experimental.pallas import tpu as pltpu

_N = 10000
_BM = 400


def _mm1_kernel(x_ref, w1_ref, s_ref):
    acc = jnp.dot(x_ref[...], w1_ref[...], preferred_element_type=jnp.float32)
    s_ref[...] = acc.astype(jnp.bfloat16)


def _mm2_kernel(w_ref, s_ref, o_ref):
    wt = w_ref[...].astype(jnp.bfloat16)
    acc = jnp.dot(wt, s_ref[...], preferred_element_type=jnp.float32)
    o_ref[...] = jnp.where(acc >= 0, acc, 0.1 * acc)


def kernel(x, w, W1):
    n, nfeat = x.shape
    nhid = W1.shape[1]

    support = pl.pallas_call(
        _mm1_kernel,
        out_shape=jax.ShapeDtypeStruct((n, nhid), jnp.bfloat16),
    )(x, W1)

    out = pl.pallas_call(
        _mm2_kernel,
        grid=(n // _BM,),
        in_specs=[
            pl.BlockSpec((_BM, n), lambda i: (i, 0)),
            pl.BlockSpec((n, nhid), lambda i: (0, 0)),
        ],
        out_specs=pl.BlockSpec((_BM, nhid), lambda i: (i, 0)),
        out_shape=jax.ShapeDtypeStruct((n, nhid), jnp.float32),
        compiler_params=pltpu.CompilerParams(
            dimension_semantics=("parallel",),
        ),
    )(w, support)
    return out
